# baseline (speedup 1.0000x reference)
import functools

import jax
import jax.numpy as jnp
from jax.experimental import pallas as pl
from jax.experimental.pallas import tpu as pltpu

N_ELEM = 56
N_SUB = 14
C = 32
CHOOSABLE_SUBS = (0, 1, 2, 3, 4, 5, 8, 11, 12)
FLOAT_MIN = -3.4e38


def _gat_block(x, adj, W, a_src, a_dst):
    bb, n, _ = x.shape
    h = jax.lax.dot_general(
        x.reshape(bb * n, x.shape[-1]), W,
        (((1,), (0,)), ((), ())), preferred_element_type=jnp.float32)
    h3 = h.reshape(bb, n, C)
    s_src = jax.lax.dot_general(
        h, a_src, (((1,), (0,)), ((), ())),
        preferred_element_type=jnp.float32).reshape(bb, n, 1)
    s_dst = jax.lax.dot_general(
        a_dst.reshape(1, C), h3, (((1,), (2,)), ((), ())),
        preferred_element_type=jnp.float32).reshape(bb, 1, n)
    e = s_src + s_dst
    e = jnp.where(e > 0, e, 0.2 * e)
    ii = jax.lax.broadcasted_iota(jnp.int32, (1, n, n), 1)
    jj = jax.lax.broadcasted_iota(jnp.int32, (1, n, n), 2)
    mask = (adj > 0) | (ii == jj)
    e = jnp.where(mask, e, -1e9)
    m = jnp.max(e, axis=-1, keepdims=True)
    p = jnp.exp(e - m)
    alpha = p / jnp.sum(p, axis=-1, keepdims=True)
    out = jax.lax.dot_general(
        alpha, h3, (((2,), (1,)), ((0,), (0,))),
        preferred_element_type=jnp.float32)
    return jnp.where(out > 0, out, jnp.exp(jnp.minimum(out, 0.0)) - 1.0)


def _fwd_kernel(obs_ref, nadj_ref, sadj_ref, wn_ref, asn_ref, adn_ref,
                ws_ref, ass_ref, ads_ref, wln_ref, chz_ref,
                ne_ref, se_ref, ch_ref):
    x = obs_ref[...]
    ne = _gat_block(x, nadj_ref[...], wn_ref[...], asn_ref[...], adn_ref[...])
    ne_ref[...] = 2.0 * ne

    bb = ne.shape[0]
    ne4 = ne.reshape(bb, N_SUB, 4, C)
    pooled = 0.25 * ((ne4[:, :, 0, :] + ne4[:, :, 1, :])
                     + (ne4[:, :, 2, :] + ne4[:, :, 3, :]))
    se = _gat_block(pooled, sadj_ref[...], ws_ref[...], ass_ref[...],
                    ads_ref[...])
    se_ref[...] = 2.0 * se

    logits = jnp.sum(se * wln_ref[...][None, :, :], axis=-1)
    logits = jnp.where(chz_ref[...] > 0, logits, FLOAT_MIN)
    rowmax = jnp.max(logits, axis=-1, keepdims=True)
    ids = jax.lax.broadcasted_iota(jnp.int32, logits.shape, 1)
    first = jnp.min(jnp.where(logits == rowmax, ids, N_SUB), axis=-1,
                    keepdims=True)
    ch_ref[...] = first


@functools.partial(jax.jit, static_argnames=("interpret",))
def _run(obs, node_adj, sub_adj, W_node, a_src_node, a_dst_node,
         W_sub, a_src_sub, a_dst_sub, W_ln, interpret=False):
    B = obs.shape[0]
    BB = 128
    grid = (B // BB,)

    chz = jnp.zeros((1, N_SUB), jnp.int32).at[0, jnp.array(CHOOSABLE_SUBS)].set(1)
    asn = a_src_node.reshape(C, 1)
    adn = a_dst_node.reshape(1, C)
    ass = a_src_sub.reshape(C, 1)
    ads = a_dst_sub.reshape(1, C)
    wln = W_ln.reshape(1, C)

    def bspec(shape):
        nd = len(shape)
        return pl.BlockSpec(shape, lambda i, _nd=nd: (i,) + (0,) * (_nd - 1))

    def rep(shape):
        nd = len(shape)
        return pl.BlockSpec(shape, lambda i, _nd=nd: (0,) * _nd)

    ne, se, choice = pl.pallas_call(
        _fwd_kernel,
        grid=grid,
        in_specs=[
            bspec((BB, N_ELEM, C)),
            bspec((BB, N_ELEM, N_ELEM)),
            bspec((BB, N_SUB, N_SUB)),
            rep((C, C)),
            rep((C, 1)),
            rep((1, C)),
            rep((C, C)),
            rep((C, 1)),
            rep((1, C)),
            rep((1, C)),
            rep((1, N_SUB)),
        ],
        out_specs=[
            bspec((BB, N_ELEM, C)),
            bspec((BB, N_SUB, C)),
            bspec((BB, 1)),
        ],
        out_shape=[
            jax.ShapeDtypeStruct((B, N_ELEM, C), jnp.float32),
            jax.ShapeDtypeStruct((B, N_SUB, C), jnp.float32),
            jax.ShapeDtypeStruct((B, 1), jnp.int32),
        ],
        interpret=interpret,
    )(obs, node_adj, sub_adj, W_node, asn, adn, W_sub, ass, ads, wln, chz)
    return ne, se, choice


def kernel(obs, node_adj, sub_adj, W_node, a_src_node, a_dst_node,
           W_sub, a_src_sub, a_dst_sub, W_ln, b_ln, elem_ids):
    del b_ln, elem_ids
    return _run(obs, node_adj, sub_adj, W_node, a_src_node, a_dst_node,
                W_sub, a_src_sub, a_dst_sub, W_ln)


# speedup vs baseline: 1.0328x; 1.0328x over previous
import functools

import jax
import jax.numpy as jnp
from jax.experimental import pallas as pl

N_ELEM = 56
N_SUB = 14
C = 32
CHOOSABLE_SUBS = (0, 1, 2, 3, 4, 5, 8, 11, 12)
FLOAT_MIN = -3.4e38


def _gat_block(x2d, mask8, W, a_src, a_dst, bb, n):
    h = jax.lax.dot_general(
        x2d, W, (((1,), (0,)), ((), ())), preferred_element_type=jnp.float32)
    h3 = h.reshape(bb, n, C)
    s_src = jax.lax.dot_general(
        h, a_src, (((1,), (0,)), ((), ())),
        preferred_element_type=jnp.float32).reshape(bb, n, 1)
    s_dst = jax.lax.dot_general(
        a_dst.reshape(1, C), h3, (((1,), (2,)), ((), ())),
        preferred_element_type=jnp.float32).reshape(bb, 1, n)
    e = s_src + s_dst
    e = jnp.maximum(e, 0.2 * e)
    e = jnp.where(mask8.astype(jnp.int32) > 0, e, -1e9)
    p = jnp.exp(e)
    r = 1.0 / jnp.sum(p, axis=-1, keepdims=True)
    alpha = p * r
    out = jax.lax.dot_general(
        alpha, h3, (((2,), (1,)), ((0,), (0,))),
        preferred_element_type=jnp.float32)
    return jnp.where(out > 0, out, jnp.exp(jnp.minimum(out, 0.0)) - 1.0)


def _fwd_kernel(obs_ref, nadj_ref, sadj_ref, wn_ref, asn_ref, adn_ref,
                ws_ref, ass_ref, ads_ref, wln_ref, chz_ref,
                ne_ref, se_ref, ch_ref):
    bb = obs_ref.shape[0]
    x2d = obs_ref[...].reshape(bb * N_ELEM, C)
    ne = _gat_block(x2d, nadj_ref[...], wn_ref[...], asn_ref[...],
                    adn_ref[...], bb, N_ELEM)
    ne_ref[...] = 2.0 * ne

    ne4 = ne.reshape(bb, N_SUB, 4, C)
    pooled = 0.25 * ((ne4[:, :, 0, :] + ne4[:, :, 1, :])
                     + (ne4[:, :, 2, :] + ne4[:, :, 3, :]))
    se = _gat_block(pooled.reshape(bb * N_SUB, C), sadj_ref[...], ws_ref[...],
                    ass_ref[...], ads_ref[...], bb, N_SUB)
    se_ref[...] = 2.0 * se

    logits = jnp.sum(se * wln_ref[...][None, :, :], axis=-1)
    logits = jnp.where(chz_ref[...] > 0, logits, FLOAT_MIN)
    rowmax = jnp.max(logits, axis=-1, keepdims=True)
    ids = jax.lax.broadcasted_iota(jnp.int32, logits.shape, 1)
    first = jnp.min(jnp.where(logits == rowmax, ids, N_SUB), axis=-1,
                    keepdims=True)
    ch_ref[...] = first


@functools.partial(jax.jit, static_argnames=("interpret",))
def _run(obs, node_adj, sub_adj, W_node, a_src_node, a_dst_node,
         W_sub, a_src_sub, a_dst_sub, W_ln, interpret=False):
    B = obs.shape[0]
    BB = 128
    grid = (B // BB,)

    obs_b = obs.astype(jnp.bfloat16)
    wn_b = W_node.astype(jnp.bfloat16)
    nmask = ((node_adj > 0) | jnp.eye(N_ELEM, dtype=bool)[None]).astype(jnp.int8)
    smask = ((sub_adj > 0) | jnp.eye(N_SUB, dtype=bool)[None]).astype(jnp.int8)

    chz = jnp.zeros((1, N_SUB), jnp.int32).at[0, jnp.array(CHOOSABLE_SUBS)].set(1)
    asn = a_src_node.reshape(C, 1)
    adn = a_dst_node.reshape(1, C)
    ass = a_src_sub.reshape(C, 1)
    ads = a_dst_sub.reshape(1, C)
    wln = W_ln.reshape(1, C)

    def bspec(shape):
        nd = len(shape)
        return pl.BlockSpec(shape, lambda i, _nd=nd: (i,) + (0,) * (_nd - 1))

    def rep(shape):
        nd = len(shape)
        return pl.BlockSpec(shape, lambda i, _nd=nd: (0,) * _nd)

    ne, se, choice = pl.pallas_call(
        _fwd_kernel,
        grid=grid,
        in_specs=[
            bspec((BB, N_ELEM, C)),
            bspec((BB, N_ELEM, N_ELEM)),
            bspec((BB, N_SUB, N_SUB)),
            rep((C, C)),
            rep((C, 1)),
            rep((1, C)),
            rep((C, C)),
            rep((C, 1)),
            rep((1, C)),
            rep((1, C)),
            rep((1, N_SUB)),
        ],
        out_specs=[
            bspec((BB, N_ELEM, C)),
            bspec((BB, N_SUB, C)),
            bspec((BB, 1)),
        ],
        out_shape=[
            jax.ShapeDtypeStruct((B, N_ELEM, C), jnp.float32),
            jax.ShapeDtypeStruct((B, N_SUB, C), jnp.float32),
            jax.ShapeDtypeStruct((B, 1), jnp.int32),
        ],
        interpret=interpret,
    )(obs_b, nmask, smask, wn_b, asn, adn, W_sub, ass, ads, wln, chz)
    return ne, se, choice


def kernel(obs, node_adj, sub_adj, W_node, a_src_node, a_dst_node,
           W_sub, a_src_sub, a_dst_sub, W_ln, b_ln, elem_ids):
    del b_ln, elem_ids
    return _run(obs, node_adj, sub_adj, W_node, a_src_node, a_dst_node,
                W_sub, a_src_sub, a_dst_sub, W_ln)


# speedup vs baseline: 1.0432x; 1.0101x over previous
import functools

import jax
import jax.numpy as jnp
from jax.experimental import pallas as pl

N_ELEM = 56
N_SUB = 14
C = 32
CHOOSABLE_SUBS = (0, 1, 2, 3, 4, 5, 8, 11, 12)
FLOAT_MIN = -3.4e38


def _gat_block(x2d, mask8, W, a_src, a_dst, bb, n):
    h = jax.lax.dot_general(
        x2d, W, (((1,), (0,)), ((), ())), preferred_element_type=jnp.float32)
    h3 = h.reshape(bb, n, C)
    s_src = jax.lax.dot_general(
        h, a_src, (((1,), (0,)), ((), ())),
        preferred_element_type=jnp.float32).reshape(bb, n, 1)
    s_dst = jax.lax.dot_general(
        a_dst.reshape(1, C), h3, (((1,), (2,)), ((), ())),
        preferred_element_type=jnp.float32).reshape(bb, 1, n)
    e = s_src + s_dst
    e = jnp.maximum(e, 0.2 * e)
    e = jnp.where(mask8.astype(jnp.int32) > 0, e, -1e9)
    p = jnp.exp(e)
    r = 1.0 / jnp.sum(p, axis=-1, keepdims=True)
    alpha = p * r
    out = jax.lax.dot_general(
        alpha, h3, (((2,), (1,)), ((0,), (0,))),
        preferred_element_type=jnp.float32)
    return jnp.where(out > 0, out, jnp.exp(jnp.minimum(out, 0.0)) - 1.0)


def _fwd_kernel(obs_ref, nadj_ref, sadj_ref, wn_ref, asn_ref, adn_ref,
                ws_ref, ass_ref, ads_ref, wln_ref, chz_ref,
                ne_ref, se_ref, ch_ref):
    bb = obs_ref.shape[0]
    x2d = obs_ref[...].reshape(bb * N_ELEM, C)
    ne = _gat_block(x2d, nadj_ref[...], wn_ref[...], asn_ref[...],
                    adn_ref[...], bb, N_ELEM)
    ne_ref[...] = 2.0 * ne

    ne4 = ne.reshape(bb, N_SUB, 4, C)
    pooled = 0.25 * ((ne4[:, :, 0, :] + ne4[:, :, 1, :])
                     + (ne4[:, :, 2, :] + ne4[:, :, 3, :]))
    se = _gat_block(pooled.reshape(bb * N_SUB, C), sadj_ref[...], ws_ref[...],
                    ass_ref[...], ads_ref[...], bb, N_SUB)
    se_ref[...] = 2.0 * se

    logits = jnp.sum(se * wln_ref[...][None, :, :], axis=-1)
    logits = jnp.where(chz_ref[...] > 0, logits, FLOAT_MIN)
    rowmax = jnp.max(logits, axis=-1, keepdims=True)
    ids = jax.lax.broadcasted_iota(jnp.int32, logits.shape, 1)
    first = jnp.min(jnp.where(logits == rowmax, ids, N_SUB), axis=-1,
                    keepdims=True)
    ch_ref[...] = first


@functools.partial(jax.jit, static_argnames=("interpret",))
def _run(obs, node_adj, sub_adj, W_node, a_src_node, a_dst_node,
         W_sub, a_src_sub, a_dst_sub, W_ln, interpret=False):
    B = obs.shape[0]
    BB = 256
    grid = (B // BB,)

    obs_b = obs.astype(jnp.bfloat16)
    wn_b = W_node.astype(jnp.bfloat16)
    nmask = ((node_adj > 0) | jnp.eye(N_ELEM, dtype=bool)[None]).astype(jnp.int8)
    smask = ((sub_adj > 0) | jnp.eye(N_SUB, dtype=bool)[None]).astype(jnp.int8)

    chz = jnp.zeros((1, N_SUB), jnp.int32).at[0, jnp.array(CHOOSABLE_SUBS)].set(1)
    asn = a_src_node.reshape(C, 1)
    adn = a_dst_node.reshape(1, C)
    ass = a_src_sub.reshape(C, 1)
    ads = a_dst_sub.reshape(1, C)
    wln = W_ln.reshape(1, C)

    def bspec(shape):
        nd = len(shape)
        return pl.BlockSpec(shape, lambda i, _nd=nd: (i,) + (0,) * (_nd - 1))

    def rep(shape):
        nd = len(shape)
        return pl.BlockSpec(shape, lambda i, _nd=nd: (0,) * _nd)

    ne, se, choice = pl.pallas_call(
        _fwd_kernel,
        grid=grid,
        in_specs=[
            bspec((BB, N_ELEM, C)),
            bspec((BB, N_ELEM, N_ELEM)),
            bspec((BB, N_SUB, N_SUB)),
            rep((C, C)),
            rep((C, 1)),
            rep((1, C)),
            rep((C, C)),
            rep((C, 1)),
            rep((1, C)),
            rep((1, C)),
            rep((1, N_SUB)),
        ],
        out_specs=[
            bspec((BB, N_ELEM, C)),
            bspec((BB, N_SUB, C)),
            bspec((BB, 1)),
        ],
        out_shape=[
            jax.ShapeDtypeStruct((B, N_ELEM, C), jnp.float32),
            jax.ShapeDtypeStruct((B, N_SUB, C), jnp.float32),
            jax.ShapeDtypeStruct((B, 1), jnp.int32),
        ],
        interpret=interpret,
    )(obs_b, nmask, smask, wn_b, asn, adn, W_sub, ass, ads, wln, chz)
    return ne, se, choice


def kernel(obs, node_adj, sub_adj, W_node, a_src_node, a_dst_node,
           W_sub, a_src_sub, a_dst_sub, W_ln, b_ln, elem_ids):
    del b_ln, elem_ids
    return _run(obs, node_adj, sub_adj, W_node, a_src_node, a_dst_node,
                W_sub, a_src_sub, a_dst_sub, W_ln)
